# baseline (device time: 74419 ns/iter reference)
import jax
import jax.numpy as jnp
from jax import lax
from jax.experimental import pallas as pl
from jax.experimental.pallas import tpu as pltpu

N_LAYERS = 3


def kernel(x, Win0, Wout0, Win1, Wout1, Win2, Wout2):
    b, d_y = x.shape
    _, h_x = Win0.shape

    def body(x_ref, win0_ref, wout0_ref, win1_ref, wout1_ref, win2_ref,
             wout2_ref, out_ref, h_send, h_recv, o_send, o_recv,
             send_sems, recv_sems):
        my_x = lax.axis_index("x")
        my_y = lax.axis_index("y")
        y_nbr = (my_x, 1 - my_y)
        x_nbr = (1 - my_x, my_y)

        barrier_sem = pltpu.get_barrier_semaphore()
        for nbr in (y_nbr, x_nbr):
            pl.semaphore_signal(
                barrier_sem, inc=1,
                device_id=nbr, device_id_type=pl.DeviceIdType.MESH,
            )
        pl.semaphore_wait(barrier_sem, 2)

        wins = (win0_ref, win1_ref, win2_ref)
        wouts = (wout0_ref, wout1_ref, wout2_ref)

        acc = x_ref[...]
        for k in range(N_LAYERS):
            part_h = jnp.dot(acc, wins[k][...],
                             preferred_element_type=jnp.float32)
            h_send[...] = part_h
            rdma_h = pltpu.make_async_remote_copy(
                src_ref=h_send,
                dst_ref=h_recv.at[k],
                send_sem=send_sems.at[0],
                recv_sem=recv_sems.at[2 * k],
                device_id=y_nbr,
                device_id_type=pl.DeviceIdType.MESH,
            )
            rdma_h.start()
            rdma_h.wait()
            h = jnp.maximum(part_h + h_recv[k], 0.0)

            part_o = jnp.dot(h, wouts[k][...],
                             preferred_element_type=jnp.float32)
            o_send[...] = part_o
            rdma_o = pltpu.make_async_remote_copy(
                src_ref=o_send,
                dst_ref=o_recv.at[k],
                send_sem=send_sems.at[1],
                recv_sem=recv_sems.at[2 * k + 1],
                device_id=x_nbr,
                device_id_type=pl.DeviceIdType.MESH,
            )
            rdma_o.start()
            rdma_o.wait()
            acc = part_o + o_recv[k]

        out_ref[...] = acc

    vmem = pl.BlockSpec(memory_space=pltpu.VMEM)
    return pl.pallas_call(
        body,
        out_shape=jax.ShapeDtypeStruct((b, d_y), jnp.float32),
        in_specs=[vmem] * 7,
        out_specs=vmem,
        scratch_shapes=[
            pltpu.VMEM((b, h_x), jnp.float32),
            pltpu.VMEM((N_LAYERS, b, h_x), jnp.float32),
            pltpu.VMEM((b, d_y), jnp.float32),
            pltpu.VMEM((N_LAYERS, b, d_y), jnp.float32),
            pltpu.SemaphoreType.DMA((2,)),
            pltpu.SemaphoreType.DMA((2 * N_LAYERS,)),
        ],
        compiler_params=pltpu.CompilerParams(collective_id=0),
    )(x, Win0, Wout0, Win1, Wout1, Win2, Wout2)


# device time: 54485 ns/iter; 1.3659x vs baseline; 1.3659x over previous
import jax
import jax.numpy as jnp
from jax import lax
from jax.experimental import pallas as pl
from jax.experimental.pallas import tpu as pltpu

N_LAYERS = 3
R = 4


def kernel(x, Win0, Wout0, Win1, Wout1, Win2, Wout2):
    b, d_y = x.shape
    _, h_x = Win0.shape
    m = b // R

    def body(x_ref, win0_ref, wout0_ref, win1_ref, wout1_ref, win2_ref,
             wout2_ref, out_ref, h_send, h_recv, o_send, o_recv,
             h_send_sems, o_send_sems, h_recv_sems, o_recv_sems):
        my_x = lax.axis_index("x")
        my_y = lax.axis_index("y")
        y_nbr = (my_x, 1 - my_y)
        x_nbr = (1 - my_x, my_y)

        barrier_sem = pltpu.get_barrier_semaphore()
        for nbr in (y_nbr, x_nbr):
            pl.semaphore_signal(
                barrier_sem, inc=1,
                device_id=nbr, device_id_type=pl.DeviceIdType.MESH,
            )
        pl.semaphore_wait(barrier_sem, 2)

        wins = (win0_ref, win1_ref, win2_ref)
        wouts = (wout0_ref, wout1_ref, wout2_ref)

        def h_rdma(k, c):
            return pltpu.make_async_remote_copy(
                src_ref=h_send.at[c],
                dst_ref=h_recv.at[k, c],
                send_sem=h_send_sems.at[c],
                recv_sem=h_recv_sems.at[k, c],
                device_id=y_nbr,
                device_id_type=pl.DeviceIdType.MESH,
            )

        def o_rdma(k, c):
            return pltpu.make_async_remote_copy(
                src_ref=o_send.at[c],
                dst_ref=o_recv.at[k, c],
                send_sem=o_send_sems.at[c],
                recv_sem=o_recv_sems.at[k, c],
                device_id=x_nbr,
                device_id_type=pl.DeviceIdType.MESH,
            )

        for k in range(N_LAYERS):
            for c in range(R):
                if k == 0:
                    acc_c = x_ref[pl.ds(c * m, m), :]
                else:
                    o_rdma(k - 1, c).wait_recv()
                    acc_c = o_send[c] + o_recv[k - 1, c]
                    h_rdma(k - 1, c).wait_send()
                h_send[c] = jnp.dot(acc_c, wins[k][...],
                                    preferred_element_type=jnp.float32)
                h_rdma(k, c).start()
            for c in range(R):
                h_rdma(k, c).wait_recv()
                h_c = jnp.maximum(h_send[c] + h_recv[k, c], 0.0)
                if k > 0:
                    o_rdma(k - 1, c).wait_send()
                o_send[c] = jnp.dot(h_c, wouts[k][...],
                                    preferred_element_type=jnp.float32)
                o_rdma(k, c).start()

        last = N_LAYERS - 1
        for c in range(R):
            o_rdma(last, c).wait_recv()
            out_ref[pl.ds(c * m, m), :] = o_send[c] + o_recv[last, c]
            h_rdma(last, c).wait_send()
            o_rdma(last, c).wait_send()

    vmem = pl.BlockSpec(memory_space=pltpu.VMEM)
    return pl.pallas_call(
        body,
        out_shape=jax.ShapeDtypeStruct((b, d_y), jnp.float32),
        in_specs=[vmem] * 7,
        out_specs=vmem,
        scratch_shapes=[
            pltpu.VMEM((R, m, h_x), jnp.float32),
            pltpu.VMEM((N_LAYERS, R, m, h_x), jnp.float32),
            pltpu.VMEM((R, m, d_y), jnp.float32),
            pltpu.VMEM((N_LAYERS, R, m, d_y), jnp.float32),
            pltpu.SemaphoreType.DMA((R,)),
            pltpu.SemaphoreType.DMA((R,)),
            pltpu.SemaphoreType.DMA((N_LAYERS, R)),
            pltpu.SemaphoreType.DMA((N_LAYERS, R)),
        ],
        compiler_params=pltpu.CompilerParams(collective_id=0),
    )(x, Win0, Wout0, Win1, Wout1, Win2, Wout2)


# device time: 50340 ns/iter; 1.4783x vs baseline; 1.0823x over previous
import jax
import jax.numpy as jnp
from jax import lax
from jax.experimental import pallas as pl
from jax.experimental.pallas import tpu as pltpu

N_LAYERS = 3
R = 4
OFFSET = 2


def kernel(x, Win0, Wout0, Win1, Wout1, Win2, Wout2):
    b, d_y = x.shape
    _, h_x = Win0.shape
    m = b // R

    def body(x_ref, win0_ref, wout0_ref, win1_ref, wout1_ref, win2_ref,
             wout2_ref, out_ref, h_send, h_recv, o_send, o_recv,
             h_send_sems, o_send_sems, h_recv_sems, o_recv_sems):
        my_x = lax.axis_index("x")
        my_y = lax.axis_index("y")
        y_nbr = (my_x, 1 - my_y)
        x_nbr = (1 - my_x, my_y)

        barrier_sem = pltpu.get_barrier_semaphore()
        for nbr in (y_nbr, x_nbr):
            pl.semaphore_signal(
                barrier_sem, inc=1,
                device_id=nbr, device_id_type=pl.DeviceIdType.MESH,
            )
        pl.semaphore_wait(barrier_sem, 2)

        wins = (win0_ref, win1_ref, win2_ref)
        wouts = (wout0_ref, wout1_ref, wout2_ref)

        def h_rdma(k, c):
            return pltpu.make_async_remote_copy(
                src_ref=h_send.at[c],
                dst_ref=h_recv.at[k, c],
                send_sem=h_send_sems.at[c],
                recv_sem=h_recv_sems.at[k, c],
                device_id=y_nbr,
                device_id_type=pl.DeviceIdType.MESH,
            )

        def o_rdma(k, c):
            return pltpu.make_async_remote_copy(
                src_ref=o_send.at[c],
                dst_ref=o_recv.at[k, c],
                send_sem=o_send_sems.at[c],
                recv_sem=o_recv_sems.at[k, c],
                device_id=x_nbr,
                device_id_type=pl.DeviceIdType.MESH,
            )

        def stage_a(k, c):
            if k == 0:
                acc_c = x_ref[pl.ds(c * m, m), :]
            else:
                o_rdma(k - 1, c).wait_recv()
                acc_c = o_send[c] + o_recv[k - 1, c]
                h_rdma(k - 1, c).wait_send()
            h_send[c] = jnp.dot(acc_c, wins[k][...],
                                preferred_element_type=jnp.float32)
            h_rdma(k, c).start()

        def stage_b(k, c):
            h_rdma(k, c).wait_recv()
            h_c = jnp.maximum(h_send[c] + h_recv[k, c], 0.0)
            if k > 0:
                o_rdma(k - 1, c).wait_send()
            o_send[c] = jnp.dot(h_c, wouts[k][...],
                                preferred_element_type=jnp.float32)
            o_rdma(k, c).start()

        for c in range(R):
            stage_a(0, c)
        for k in range(N_LAYERS):
            for c in range(R):
                stage_b(k, c)
                if k + 1 < N_LAYERS and c >= OFFSET:
                    stage_a(k + 1, c - OFFSET)
            if k + 1 < N_LAYERS:
                for c in range(R - OFFSET, R):
                    stage_a(k + 1, c)

        last = N_LAYERS - 1
        for c in range(R):
            o_rdma(last, c).wait_recv()
            out_ref[pl.ds(c * m, m), :] = o_send[c] + o_recv[last, c]
            h_rdma(last, c).wait_send()
            o_rdma(last, c).wait_send()

    vmem = pl.BlockSpec(memory_space=pltpu.VMEM)
    return pl.pallas_call(
        body,
        out_shape=jax.ShapeDtypeStruct((b, d_y), jnp.float32),
        in_specs=[vmem] * 7,
        out_specs=vmem,
        scratch_shapes=[
            pltpu.VMEM((R, m, h_x), jnp.float32),
            pltpu.VMEM((N_LAYERS, R, m, h_x), jnp.float32),
            pltpu.VMEM((R, m, d_y), jnp.float32),
            pltpu.VMEM((N_LAYERS, R, m, d_y), jnp.float32),
            pltpu.SemaphoreType.DMA((R,)),
            pltpu.SemaphoreType.DMA((R,)),
            pltpu.SemaphoreType.DMA((N_LAYERS, R)),
            pltpu.SemaphoreType.DMA((N_LAYERS, R)),
        ],
        compiler_params=pltpu.CompilerParams(collective_id=0),
    )(x, Win0, Wout0, Win1, Wout1, Win2, Wout2)


# device time: 49831 ns/iter; 1.4934x vs baseline; 1.0102x over previous
import jax
import jax.numpy as jnp
from jax import lax
from jax.experimental import pallas as pl
from jax.experimental.pallas import tpu as pltpu

N_LAYERS = 3
R = 8
OFFSET = 2


def kernel(x, Win0, Wout0, Win1, Wout1, Win2, Wout2):
    b, d_y = x.shape
    _, h_x = Win0.shape
    m = b // R

    def body(x_ref, win0_ref, wout0_ref, win1_ref, wout1_ref, win2_ref,
             wout2_ref, out_ref, h_send, h_recv, o_send, o_recv,
             h_send_sems, o_send_sems, h_recv_sems, o_recv_sems):
        my_x = lax.axis_index("x")
        my_y = lax.axis_index("y")
        y_nbr = (my_x, 1 - my_y)
        x_nbr = (1 - my_x, my_y)

        barrier_sem = pltpu.get_barrier_semaphore()
        for nbr in (y_nbr, x_nbr):
            pl.semaphore_signal(
                barrier_sem, inc=1,
                device_id=nbr, device_id_type=pl.DeviceIdType.MESH,
            )
        pl.semaphore_wait(barrier_sem, 2)

        wins = (win0_ref, win1_ref, win2_ref)
        wouts = (wout0_ref, wout1_ref, wout2_ref)

        def h_rdma(k, c):
            return pltpu.make_async_remote_copy(
                src_ref=h_send.at[c],
                dst_ref=h_recv.at[k, c],
                send_sem=h_send_sems.at[c],
                recv_sem=h_recv_sems.at[k, c],
                device_id=y_nbr,
                device_id_type=pl.DeviceIdType.MESH,
            )

        def o_rdma(k, c):
            return pltpu.make_async_remote_copy(
                src_ref=o_send.at[c],
                dst_ref=o_recv.at[k, c],
                send_sem=o_send_sems.at[c],
                recv_sem=o_recv_sems.at[k, c],
                device_id=x_nbr,
                device_id_type=pl.DeviceIdType.MESH,
            )

        def stage_a(k, c):
            if k == 0:
                acc_c = x_ref[pl.ds(c * m, m), :]
            else:
                o_rdma(k - 1, c).wait_recv()
                acc_c = o_send[c] + o_recv[k - 1, c]
                h_rdma(k - 1, c).wait_send()
            h_send[c] = jnp.dot(acc_c, wins[k][...],
                                preferred_element_type=jnp.float32)
            h_rdma(k, c).start()

        def stage_b(k, c):
            h_rdma(k, c).wait_recv()
            h_c = jnp.maximum(h_send[c] + h_recv[k, c], 0.0)
            if k > 0:
                o_rdma(k - 1, c).wait_send()
            o_send[c] = jnp.dot(h_c, wouts[k][...],
                                preferred_element_type=jnp.float32)
            o_rdma(k, c).start()

        for c in range(R):
            stage_a(0, c)
        for k in range(N_LAYERS):
            for c in range(R):
                stage_b(k, c)
                if k + 1 < N_LAYERS and c >= OFFSET:
                    stage_a(k + 1, c - OFFSET)
            if k + 1 < N_LAYERS:
                for c in range(R - OFFSET, R):
                    stage_a(k + 1, c)

        last = N_LAYERS - 1
        for c in range(R):
            o_rdma(last, c).wait_recv()
            out_ref[pl.ds(c * m, m), :] = o_send[c] + o_recv[last, c]
            h_rdma(last, c).wait_send()
            o_rdma(last, c).wait_send()

    vmem = pl.BlockSpec(memory_space=pltpu.VMEM)
    return pl.pallas_call(
        body,
        out_shape=jax.ShapeDtypeStruct((b, d_y), jnp.float32),
        in_specs=[vmem] * 7,
        out_specs=vmem,
        scratch_shapes=[
            pltpu.VMEM((R, m, h_x), jnp.float32),
            pltpu.VMEM((N_LAYERS, R, m, h_x), jnp.float32),
            pltpu.VMEM((R, m, d_y), jnp.float32),
            pltpu.VMEM((N_LAYERS, R, m, d_y), jnp.float32),
            pltpu.SemaphoreType.DMA((R,)),
            pltpu.SemaphoreType.DMA((R,)),
            pltpu.SemaphoreType.DMA((N_LAYERS, R)),
            pltpu.SemaphoreType.DMA((N_LAYERS, R)),
        ],
        compiler_params=pltpu.CompilerParams(collective_id=0),
    )(x, Win0, Wout0, Win1, Wout1, Win2, Wout2)


# device time: 34837 ns/iter; 2.1362x vs baseline; 1.4304x over previous
import jax
import jax.numpy as jnp
from jax import lax
from jax.experimental import pallas as pl
from jax.experimental.pallas import tpu as pltpu

N_LAYERS = 3
R = 4
OFFSET = 1
COMM_DTYPE = jnp.bfloat16


def kernel(x, Win0, Wout0, Win1, Wout1, Win2, Wout2):
    b, d_y = x.shape
    _, h_x = Win0.shape
    m = b // R

    def body(x_ref, win0_ref, wout0_ref, win1_ref, wout1_ref, win2_ref,
             wout2_ref, out_ref, h_send, h_recv, o_send, o_recv,
             h_send_sems, o_send_sems, h_recv_sems, o_recv_sems):
        my_x = lax.axis_index("x")
        my_y = lax.axis_index("y")
        y_nbr = (my_x, 1 - my_y)
        x_nbr = (1 - my_x, my_y)

        barrier_sem = pltpu.get_barrier_semaphore()
        for nbr in (y_nbr, x_nbr):
            pl.semaphore_signal(
                barrier_sem, inc=1,
                device_id=nbr, device_id_type=pl.DeviceIdType.MESH,
            )
        pl.semaphore_wait(barrier_sem, 2)

        wins = (win0_ref, win1_ref, win2_ref)
        wouts = (wout0_ref, wout1_ref, wout2_ref)

        part_h = [None] * R
        part_o = [None] * R

        def h_rdma(k, c):
            return pltpu.make_async_remote_copy(
                src_ref=h_send.at[c],
                dst_ref=h_recv.at[k, c],
                send_sem=h_send_sems.at[c],
                recv_sem=h_recv_sems.at[k, c],
                device_id=y_nbr,
                device_id_type=pl.DeviceIdType.MESH,
            )

        def o_rdma(k, c):
            return pltpu.make_async_remote_copy(
                src_ref=o_send.at[c],
                dst_ref=o_recv.at[k, c],
                send_sem=o_send_sems.at[c],
                recv_sem=o_recv_sems.at[k, c],
                device_id=x_nbr,
                device_id_type=pl.DeviceIdType.MESH,
            )

        def stage_a(k, c):
            if k == 0:
                acc_c = x_ref[pl.ds(c * m, m), :]
            else:
                o_rdma(k - 1, c).wait_recv()
                acc_c = part_o[c] + o_recv[k - 1, c].astype(jnp.float32)
            part_h[c] = jnp.dot(acc_c, wins[k][...],
                                preferred_element_type=jnp.float32)
            if k > 0:
                h_rdma(k - 1, c).wait_send()
            h_send[c] = part_h[c].astype(COMM_DTYPE)
            h_rdma(k, c).start()

        def stage_b(k, c):
            h_rdma(k, c).wait_recv()
            h_c = jnp.maximum(
                part_h[c] + h_recv[k, c].astype(jnp.float32), 0.0)
            part_o[c] = jnp.dot(h_c, wouts[k][...],
                                preferred_element_type=jnp.float32)
            if k > 0:
                o_rdma(k - 1, c).wait_send()
            o_send[c] = part_o[c].astype(COMM_DTYPE)
            o_rdma(k, c).start()

        for c in range(R):
            stage_a(0, c)
        for k in range(N_LAYERS):
            for c in range(R):
                stage_b(k, c)
                if k + 1 < N_LAYERS and c >= OFFSET:
                    stage_a(k + 1, c - OFFSET)
            if k + 1 < N_LAYERS:
                for c in range(R - OFFSET, R):
                    stage_a(k + 1, c)

        last = N_LAYERS - 1
        for c in range(R):
            o_rdma(last, c).wait_recv()
            out_ref[pl.ds(c * m, m), :] = (
                part_o[c] + o_recv[last, c].astype(jnp.float32))
            h_rdma(last, c).wait_send()
            o_rdma(last, c).wait_send()

    vmem = pl.BlockSpec(memory_space=pltpu.VMEM)
    return pl.pallas_call(
        body,
        out_shape=jax.ShapeDtypeStruct((b, d_y), jnp.float32),
        in_specs=[vmem] * 7,
        out_specs=vmem,
        scratch_shapes=[
            pltpu.VMEM((R, m, h_x), COMM_DTYPE),
            pltpu.VMEM((N_LAYERS, R, m, h_x), COMM_DTYPE),
            pltpu.VMEM((R, m, d_y), COMM_DTYPE),
            pltpu.VMEM((N_LAYERS, R, m, d_y), COMM_DTYPE),
            pltpu.SemaphoreType.DMA((R,)),
            pltpu.SemaphoreType.DMA((R,)),
            pltpu.SemaphoreType.DMA((N_LAYERS, R)),
            pltpu.SemaphoreType.DMA((N_LAYERS, R)),
        ],
        compiler_params=pltpu.CompilerParams(collective_id=0),
    )(x, Win0, Wout0, Win1, Wout1, Win2, Wout2)
